# baseline (speedup 1.0000x reference)
import jax
import jax.numpy as jnp
from jax import lax
from jax.experimental import pallas as pl
from jax.experimental.pallas import tpu as pltpu
from jax.experimental.pallas import tpu_sc as plsc

N = 10000
NP = 10240
E = 160000
EP = 163840
C = 256
H = 128
EPS = 1e-5

_MESH = plsc.VectorSubcoreMesh(core_axis_name="c", subcore_axis_name="s")


def _deg_body(dst_hbm, zeros_hbm, ones_hbm, outa_hbm, outb_hbm, idx_v, ones_v,
              zb_v, db_v, deg_sh):
  c = lax.axis_index("c")
  s = lax.axis_index("s")
  w = c * 16 + s
  pltpu.sync_copy(dst_hbm.at[w], idx_v)
  pltpu.sync_copy(ones_hbm, ones_v)
  pltpu.sync_copy(zeros_hbm, zb_v)
  pltpu.sync_copy(zb_v, deg_sh.at[pl.ds(s * 640, 640)])
  plsc.subcore_barrier()

  def step(k, carry):
    pltpu.sync_copy(ones_v, deg_sh.at[idx_v.at[k]], add=True)
    return carry

  lax.fori_loop(0, 40, step, 0)
  plsc.subcore_barrier()
  pltpu.sync_copy(deg_sh.at[pl.ds(s * 640, 640)], db_v)
  for ci, o_hbm in ((0, outa_hbm), (1, outb_hbm)):
    @pl.when(c == ci)
    def _():
      pltpu.sync_copy(db_v, o_hbm.at[pl.ds(s * 640, 640)])


@jax.jit
def _deg_call(dst_k1, zeros640, ones128):
  return pl.kernel(
      _deg_body,
      out_type=(jax.ShapeDtypeStruct((NP,), jnp.float32),
                jax.ShapeDtypeStruct((NP,), jnp.float32)),
      mesh=_MESH,
      scratch_types=[
          pltpu.VMEM((40, 128), jnp.int32),
          pltpu.VMEM((128,), jnp.float32),
          pltpu.VMEM((640,), jnp.float32),
          pltpu.VMEM((640,), jnp.float32),
          pltpu.VMEM_SHARED((NP,), jnp.float32),
      ],
  )(dst_k1, zeros640, ones128)


def _lin_body(x_ref, w_ref, dega_ref, degb_ref, xs0_ref, xs1_ref, dis_ref):
  xw = lax.dot_general(x_ref[...], w_ref[...], (((1,), (1,)), ((), ())),
                       preferred_element_type=jnp.float32)
  deg = dega_ref[...] + degb_ref[...] + 1.0
  dis = lax.rsqrt(deg)
  xs = xw * dis
  xs0_ref[...] = xs[:, :H]
  xs1_ref[...] = xs[:, H:]
  dis_ref[...] = dis


@jax.jit
def _lin_call(x, W, dega, degb):
  return pl.pallas_call(
      _lin_body,
      grid=(16,),
      in_specs=[
          pl.BlockSpec((640, C), lambda i: (i, 0)),
          pl.BlockSpec((C, C), lambda i: (0, 0)),
          pl.BlockSpec((640, 1), lambda i: (i, 0)),
          pl.BlockSpec((640, 1), lambda i: (i, 0)),
      ],
      out_specs=[
          pl.BlockSpec((640, H), lambda i: (i, 0)),
          pl.BlockSpec((640, H), lambda i: (i, 0)),
          pl.BlockSpec((640, 1), lambda i: (i, 0)),
      ],
      out_shape=[
          jax.ShapeDtypeStruct((NP, H), jnp.float32),
          jax.ShapeDtypeStruct((NP, H), jnp.float32),
          jax.ShapeDtypeStruct((NP, 1), jnp.float32),
      ],
  )(x, W, dega, degb)


def _bounce(get_src, get_dst, rows, sgs, sss, n):
  gd = [None, None]
  sd = [None, None]
  for j in range(n):
    p = j % 2
    if sd[p] is not None:
      sd[p].wait()
      sd[p] = None
    gd[p] = pltpu.async_copy(get_src(j), rows[p], sgs[p])
    if j > 0:
      gd[1 - p].wait()
      sd[1 - p] = pltpu.async_copy(rows[1 - p], get_dst(j - 1), sss[1 - p])
  q = (n - 1) % 2
  gd[q].wait()
  sd[q] = pltpu.async_copy(rows[q], get_dst(n - 1), sss[q])
  sd[0].wait()
  sd[1].wait()


_NB = 6
_CH = 40
_SEG = 5
_CPS = 50


def _agg_body(src_hbm, dst_hbm, xs0_hbm, xs1_hbm, out_hbm, sidx_v, didx_v,
              *rest):
  rows = rest[:_NB]
  sgs = rest[_NB:2 * _NB]
  sss = rest[2 * _NB:3 * _NB]
  agg_sh = rest[3 * _NB]
  c = lax.axis_index("c")
  s = lax.axis_index("s")

  for ci, xs_hbm in ((0, xs0_hbm), (1, xs1_hbm)):
    @pl.when(c == ci)
    def _():
      _bounce(lambda j: xs_hbm.at[pl.ds(s * 640 + j * _CH, _CH)],
              lambda j: agg_sh.at[pl.ds(s * 640 + j * _CH, _CH)],
              rows, sgs, sss, 640 // _CH)

  plsc.subcore_barrier()

  pf = _NB - 1
  for ci, xs_hbm in ((0, xs0_hbm), (1, xs1_hbm)):
    @pl.when(c == ci)
    def _():
      def seg(f, carry):
        pltpu.sync_copy(src_hbm.at[s * _SEG + f], sidx_v)
        pltpu.sync_copy(dst_hbm.at[s * _SEG + f], didx_v)
        gd = [None] * _NB
        sd = [None] * _NB
        for k in range(pf):
          gd[k] = pltpu.async_copy(xs_hbm.at[sidx_v.at[k]], rows[k], sgs[k])
        for k in range(_CPS):
          p = k % _NB
          gd[p].wait()
          if k + pf < _CPS:
            q = (k + pf) % _NB
            if sd[q] is not None:
              sd[q].wait()
              sd[q] = None
            gd[q] = pltpu.async_copy(
                xs_hbm.at[sidx_v.at[k + pf]], rows[q], sgs[q])
          sd[p] = pltpu.async_copy(
              rows[p], agg_sh.at[didx_v.at[k]], sss[p], add=True)
        for q in range(_NB):
          if sd[q] is not None:
            sd[q].wait()
        return carry

      lax.fori_loop(0, _SEG, seg, 0)

  plsc.subcore_barrier()
  _bounce(lambda j: agg_sh.at[pl.ds(s * 640 + j * _CH, _CH)],
          lambda j: out_hbm.at[c, pl.ds(s * 640 + j * _CH, _CH)],
          rows, sgs, sss, 640 // _CH)


@jax.jit
def _agg_call(src_k3, dst_k3, xs0, xs1):
  return pl.kernel(
      _agg_body,
      out_type=jax.ShapeDtypeStruct((2, NP, H), jnp.float32),
      mesh=_MESH,
      scratch_types=(
          [pltpu.VMEM((_CPS, _CH), jnp.int32),
           pltpu.VMEM((_CPS, _CH), jnp.int32)]
          + [pltpu.VMEM((_CH, H), jnp.float32)] * _NB
          + [pltpu.SemaphoreType.DMA] * (2 * _NB)
          + [pltpu.VMEM_SHARED((NP, H), jnp.float32)]
      ),
  )(src_k3, dst_k3, xs0, xs1)


def _bn_body(agg_ref, dis_ref, b_ref, g_ref, be_ref, y_ref, sum_s, sq_s):
  p = pl.program_id(1)
  i = pl.program_id(2)
  o = agg_ref[0] * dis_ref[...] + b_ref[0]

  @pl.when(p == 0)
  def _():
    row = i * 1024 + lax.broadcasted_iota(jnp.int32, (1024, 1), 0)
    om = jnp.where(row < N, o, 0.0)

    @pl.when(i == 0)
    def _():
      sum_s[...] = jnp.zeros((1, H), jnp.float32)
      sq_s[...] = jnp.zeros((1, H), jnp.float32)

    sum_s[...] += jnp.sum(om, axis=0, keepdims=True)
    sq_s[...] += jnp.sum(om * om, axis=0, keepdims=True)
    y_ref[...] = jnp.zeros((1024, H), jnp.float32)

  @pl.when(p == 1)
  def _():
    mean = sum_s[...] * (1.0 / N)
    var = sq_s[...] * (1.0 / N) - mean * mean
    xn = (o - mean) * lax.rsqrt(var + EPS)
    y_ref[...] = jnp.maximum(g_ref[0] * xn + be_ref[0], 0.0)


@jax.jit
def _bn_call(agg, dis, b2, g2, be2):
  return pl.pallas_call(
      _bn_body,
      grid=(2, 2, 10),
      in_specs=[
          pl.BlockSpec((1, 1024, H), lambda h, p, i: (h, i, 0)),
          pl.BlockSpec((1024, 1), lambda h, p, i: (i, 0)),
          pl.BlockSpec((1, 1, H), lambda h, p, i: (h, 0, 0)),
          pl.BlockSpec((1, 1, H), lambda h, p, i: (h, 0, 0)),
          pl.BlockSpec((1, 1, H), lambda h, p, i: (h, 0, 0)),
      ],
      out_specs=pl.BlockSpec((1024, H), lambda h, p, i: (i, h)),
      out_shape=jax.ShapeDtypeStruct((N, C), jnp.float32),
      scratch_shapes=[
          pltpu.VMEM((1, H), jnp.float32),
          pltpu.VMEM((1, H), jnp.float32),
      ],
  )(agg, dis, b2, g2, be2)


def kernel(x, edge_index, W, b, gamma, beta):
  ei = edge_index.astype(jnp.int32)
  src = ei[0]
  dst = ei[1]
  pad_ids = N + (jnp.arange(EP - E, dtype=jnp.int32) % (NP - N))
  dst_k1 = jnp.concatenate([dst, pad_ids]).reshape(32, 40, 128)
  src_k3 = src.reshape(16 * _SEG, _CPS, _CH)
  dst_k3 = dst.reshape(16 * _SEG, _CPS, _CH)

  dega, degb = _deg_call(dst_k1, jnp.zeros((640,), jnp.float32),
                         jnp.ones((128,), jnp.float32))
  xs0, xs1, dis = _lin_call(x, W, dega.reshape(NP, 1), degb.reshape(NP, 1))
  agg = _agg_call(src_k3, dst_k3, xs0, xs1)

  b2 = b.reshape(2, 1, H)
  g2 = gamma.reshape(2, 1, H)
  be2 = beta.reshape(2, 1, H)
  return _bn_call(agg, dis, b2, g2, be2)


# speedup vs baseline: 1.0436x; 1.0436x over previous
import jax
import jax.numpy as jnp
from jax import lax
from jax.experimental import pallas as pl
from jax.experimental.pallas import tpu as pltpu
from jax.experimental.pallas import tpu_sc as plsc

N = 10000
NP = 10240
E = 160000
EP = 163840
C = 256
H = 128
EPS = 1e-5

_MESH = plsc.VectorSubcoreMesh(core_axis_name="c", subcore_axis_name="s")


def _deg_body(dst_hbm, zeros_hbm, ones_hbm, outa_hbm, outb_hbm, idx_v, ones_v,
              zb_v, db_v, deg_sh):
  c = lax.axis_index("c")
  s = lax.axis_index("s")
  w = c * 16 + s
  pltpu.sync_copy(dst_hbm.at[w], idx_v)
  pltpu.sync_copy(ones_hbm, ones_v)
  pltpu.sync_copy(zeros_hbm, zb_v)
  pltpu.sync_copy(zb_v, deg_sh.at[pl.ds(s * 640, 640)])
  plsc.subcore_barrier()

  def step(k, carry):
    pltpu.sync_copy(ones_v, deg_sh.at[idx_v.at[k]], add=True)
    return carry

  lax.fori_loop(0, 40, step, 0)
  plsc.subcore_barrier()
  pltpu.sync_copy(deg_sh.at[pl.ds(s * 640, 640)], db_v)
  for ci, o_hbm in ((0, outa_hbm), (1, outb_hbm)):
    @pl.when(c == ci)
    def _():
      pltpu.sync_copy(db_v, o_hbm.at[pl.ds(s * 640, 640)])


@jax.jit
def _deg_call(dst_k1, zeros640, ones128):
  return pl.kernel(
      _deg_body,
      out_type=(jax.ShapeDtypeStruct((NP,), jnp.float32),
                jax.ShapeDtypeStruct((NP,), jnp.float32)),
      mesh=_MESH,
      scratch_types=[
          pltpu.VMEM((40, 128), jnp.int32),
          pltpu.VMEM((128,), jnp.float32),
          pltpu.VMEM((640,), jnp.float32),
          pltpu.VMEM((640,), jnp.float32),
          pltpu.VMEM_SHARED((NP,), jnp.float32),
      ],
  )(dst_k1, zeros640, ones128)


def _lin_body(x_ref, w_ref, dega_ref, degb_ref, xs0_ref, xs1_ref, dis_ref):
  xw = lax.dot_general(x_ref[...], w_ref[...], (((1,), (1,)), ((), ())),
                       preferred_element_type=jnp.float32)
  deg = dega_ref[...] + degb_ref[...] + 1.0
  dis = lax.rsqrt(deg)
  xs = xw * dis
  xs0_ref[...] = xs[:, :H]
  xs1_ref[...] = xs[:, H:]
  dis_ref[...] = dis


@jax.jit
def _lin_call(x, W, dega, degb):
  return pl.pallas_call(
      _lin_body,
      grid=(16,),
      in_specs=[
          pl.BlockSpec((640, C), lambda i: (i, 0)),
          pl.BlockSpec((C, C), lambda i: (0, 0)),
          pl.BlockSpec((640, 1), lambda i: (i, 0)),
          pl.BlockSpec((640, 1), lambda i: (i, 0)),
      ],
      out_specs=[
          pl.BlockSpec((640, H), lambda i: (i, 0)),
          pl.BlockSpec((640, H), lambda i: (i, 0)),
          pl.BlockSpec((640, 1), lambda i: (i, 0)),
      ],
      out_shape=[
          jax.ShapeDtypeStruct((NP, H), jnp.float32),
          jax.ShapeDtypeStruct((NP, H), jnp.float32),
          jax.ShapeDtypeStruct((NP, 1), jnp.float32),
      ],
  )(x, W, dega, degb)


def _bounce(get_src, get_dst, rows, sgs, sss, n):
  gd = [None, None]
  sd = [None, None]
  for j in range(n):
    p = j % 2
    if sd[p] is not None:
      sd[p].wait()
      sd[p] = None
    gd[p] = pltpu.async_copy(get_src(j), rows[p], sgs[p])
    if j > 0:
      gd[1 - p].wait()
      sd[1 - p] = pltpu.async_copy(rows[1 - p], get_dst(j - 1), sss[1 - p])
  q = (n - 1) % 2
  gd[q].wait()
  sd[q] = pltpu.async_copy(rows[q], get_dst(n - 1), sss[q])
  sd[0].wait()
  sd[1].wait()


_NB = 4
_CH = 80
_SEG = 5
_CPS = 25


def _agg_body(src_hbm, dst_hbm, xs0_hbm, xs1_hbm, out_hbm, sidx_v, didx_v,
              *rest):
  rows = rest[:_NB]
  sgs = rest[_NB:2 * _NB]
  sss = rest[2 * _NB:3 * _NB]
  agg_sh = rest[3 * _NB]
  c = lax.axis_index("c")
  s = lax.axis_index("s")

  for ci, xs_hbm in ((0, xs0_hbm), (1, xs1_hbm)):
    @pl.when(c == ci)
    def _():
      _bounce(lambda j: xs_hbm.at[pl.ds(s * 640 + j * _CH, _CH)],
              lambda j: agg_sh.at[pl.ds(s * 640 + j * _CH, _CH)],
              rows, sgs, sss, 640 // _CH)

  plsc.subcore_barrier()

  pf = _NB - 1
  for ci, xs_hbm in ((0, xs0_hbm), (1, xs1_hbm)):
    @pl.when(c == ci)
    def _():
      def seg(f, carry):
        pltpu.sync_copy(src_hbm.at[s * _SEG + f], sidx_v)
        pltpu.sync_copy(dst_hbm.at[s * _SEG + f], didx_v)
        gd = [None] * _NB
        sd = [None] * _NB
        for k in range(pf):
          gd[k] = pltpu.async_copy(xs_hbm.at[sidx_v.at[k]], rows[k], sgs[k])
        for k in range(_CPS):
          p = k % _NB
          gd[p].wait()
          if k + pf < _CPS:
            q = (k + pf) % _NB
            if sd[q] is not None:
              sd[q].wait()
              sd[q] = None
            gd[q] = pltpu.async_copy(
                xs_hbm.at[sidx_v.at[k + pf]], rows[q], sgs[q])
          sd[p] = pltpu.async_copy(
              rows[p], agg_sh.at[didx_v.at[k]], sss[p], add=True)
        for q in range(_NB):
          if sd[q] is not None:
            sd[q].wait()
        return carry

      lax.fori_loop(0, _SEG, seg, 0)

  plsc.subcore_barrier()
  _bounce(lambda j: agg_sh.at[pl.ds(s * 640 + j * _CH, _CH)],
          lambda j: out_hbm.at[c, pl.ds(s * 640 + j * _CH, _CH)],
          rows, sgs, sss, 640 // _CH)


@jax.jit
def _agg_call(src_k3, dst_k3, xs0, xs1):
  return pl.kernel(
      _agg_body,
      out_type=jax.ShapeDtypeStruct((2, NP, H), jnp.float32),
      mesh=_MESH,
      scratch_types=(
          [pltpu.VMEM((_CPS, _CH), jnp.int32),
           pltpu.VMEM((_CPS, _CH), jnp.int32)]
          + [pltpu.VMEM((_CH, H), jnp.float32)] * _NB
          + [pltpu.SemaphoreType.DMA] * (2 * _NB)
          + [pltpu.VMEM_SHARED((NP, H), jnp.float32)]
      ),
  )(src_k3, dst_k3, xs0, xs1)


def _bn_body(agg_ref, dis_ref, b_ref, g_ref, be_ref, y_ref, sum_s, sq_s):
  p = pl.program_id(1)
  i = pl.program_id(2)
  o = agg_ref[0] * dis_ref[...] + b_ref[0]

  @pl.when(p == 0)
  def _():
    row = i * 1024 + lax.broadcasted_iota(jnp.int32, (1024, 1), 0)
    om = jnp.where(row < N, o, 0.0)

    @pl.when(i == 0)
    def _():
      sum_s[...] = jnp.zeros((1, H), jnp.float32)
      sq_s[...] = jnp.zeros((1, H), jnp.float32)

    sum_s[...] += jnp.sum(om, axis=0, keepdims=True)
    sq_s[...] += jnp.sum(om * om, axis=0, keepdims=True)

  @pl.when(p == 1)
  def _():
    mean = sum_s[...] * (1.0 / N)
    var = sq_s[...] * (1.0 / N) - mean * mean
    xn = (o - mean) * lax.rsqrt(var + EPS)
    y_ref[...] = jnp.maximum(g_ref[0] * xn + be_ref[0], 0.0)


@jax.jit
def _bn_call(agg, dis, b2, g2, be2):
  return pl.pallas_call(
      _bn_body,
      grid=(2, 2, 10),
      in_specs=[
          pl.BlockSpec((1, 1024, H), lambda h, p, i: (h, i, 0)),
          pl.BlockSpec((1024, 1), lambda h, p, i: (i, 0)),
          pl.BlockSpec((1, 1, H), lambda h, p, i: (h, 0, 0)),
          pl.BlockSpec((1, 1, H), lambda h, p, i: (h, 0, 0)),
          pl.BlockSpec((1, 1, H), lambda h, p, i: (h, 0, 0)),
      ],
      out_specs=pl.BlockSpec(
          (1024, H), lambda h, p, i: (jnp.where(p == 1, i, 0), h)),
      out_shape=jax.ShapeDtypeStruct((N, C), jnp.float32),
      scratch_shapes=[
          pltpu.VMEM((1, H), jnp.float32),
          pltpu.VMEM((1, H), jnp.float32),
      ],
  )(agg, dis, b2, g2, be2)


def kernel(x, edge_index, W, b, gamma, beta):
  ei = edge_index.astype(jnp.int32)
  src = ei[0]
  dst = ei[1]
  pad_ids = N + (jnp.arange(EP - E, dtype=jnp.int32) % (NP - N))
  dst_k1 = jnp.concatenate([dst, pad_ids]).reshape(32, 40, 128)
  src_k3 = src.reshape(16 * _SEG, _CPS, _CH)
  dst_k3 = dst.reshape(16 * _SEG, _CPS, _CH)

  dega, degb = _deg_call(dst_k1, jnp.zeros((640,), jnp.float32),
                         jnp.ones((128,), jnp.float32))
  xs0, xs1, dis = _lin_call(x, W, dega.reshape(NP, 1), degb.reshape(NP, 1))
  agg = _agg_call(src_k3, dst_k3, xs0, xs1)

  b2 = b.reshape(2, 1, H)
  g2 = gamma.reshape(2, 1, H)
  be2 = beta.reshape(2, 1, H)
  return _bn_call(agg, dis, b2, g2, be2)
